# baseline (speedup 1.0000x reference)
import functools

import jax
import jax.numpy as jnp
from jax import lax
from jax.experimental import pallas as pl
from jax.experimental.pallas import tpu as pltpu
from jax.experimental.pallas import tpu_sc as plsc

B, N, C, H, W = 2, 5, 2, 1024, 1024
P = H * W
G = B * N
NW = 32
PPW = P // NW
CH = 16384
NCH = PPW // CH
L = 16

_mesh = plsc.VectorSubcoreMesh(core_axis_name="c", subcore_axis_name="s")


@functools.partial(
    pl.kernel,
    out_type=(
        jax.ShapeDtypeStruct((G * P,), jnp.float32),
        jax.ShapeDtypeStruct((G * P,), jnp.float32),
        jax.ShapeDtypeStruct((NW, L), jnp.float32),
    ),
    mesh=_mesh,
    scratch_types=[
        pltpu.VMEM((CH,), jnp.float32),
        pltpu.VMEM((CH,), jnp.float32),
        pltpu.VMEM((CH,), jnp.float32),
        pltpu.VMEM((CH,), jnp.float32),
        pltpu.VMEM((L,), jnp.float32),
    ],
)
def _sc_comm(x_hbm, maps_hbm, masks_hbm, part_hbm,
             ch0_v, ch1_v, om_v, mk_v, acc_v):
    wid = lax.axis_index("s") * 2 + lax.axis_index("c")
    base = wid * PPW
    ones = jnp.ones((L,), jnp.float32)
    zeros = jnp.zeros((L,), jnp.float32)
    half = jnp.full((L,), 0.5, jnp.float32)
    acc = zeros

    for g in range(G):
        agent = g % N
        is_even = (agent % 2) == 0
        is_rate = agent == 0
        obase = g * P + base
        i0 = (2 * g + 0) * P + base
        i1 = (2 * g + 1) * P + base

        def chunk_body(ci, acc, i0=i0, i1=i1, obase=obase,
                       is_even=is_even, is_rate=is_rate):
            off = ci * CH
            pltpu.sync_copy(x_hbm.at[pl.ds(i0 + off, CH)], ch0_v)
            pltpu.sync_copy(x_hbm.at[pl.ds(i1 + off, CH)], ch1_v)

            def vec_body(i, acc):
                sl = pl.ds(i * L, L)
                m = jnp.maximum(ch0_v[sl], ch1_v[sl])
                s = ones / (ones + jnp.exp(-m))
                mb = jnp.where(s > half, ones, zeros)
                if is_even:
                    om_v[sl] = s
                    mk_v[sl] = ones
                else:
                    om_v[sl] = s * mb
                    mk_v[sl] = mb
                if is_rate:
                    acc = acc + mb
                return acc

            acc = lax.fori_loop(0, CH // L, vec_body, acc)
            pltpu.sync_copy(om_v, maps_hbm.at[pl.ds(obase + off, CH)])
            pltpu.sync_copy(mk_v, masks_hbm.at[pl.ds(obase + off, CH)])
            return acc

        acc = lax.fori_loop(0, NCH, chunk_body, acc)

    acc_v[...] = acc
    pltpu.sync_copy(acc_v, part_hbm.at[wid])


def kernel(batch_confidence_maps, record_len, pairwise_t_matrix):
    x = batch_confidence_maps.reshape(G * C * P)
    maps_flat, masks_flat, partials = _sc_comm(x)
    comm_maps = maps_flat.reshape(B, N, 1, H, W)
    comm_masks = masks_flat.reshape(G, 1, H, W)
    rate = jnp.sum(partials) / (B * H * W)
    return (comm_maps, comm_masks, rate)


# speedup vs baseline: 1.3476x; 1.3476x over previous
import functools

import jax
import jax.numpy as jnp
from jax import lax
from jax.experimental import pallas as pl
from jax.experimental.pallas import tpu as pltpu
from jax.experimental.pallas import tpu_sc as plsc

B, N, C, H, W = 2, 5, 2, 1024, 1024
P = H * W
G = B * N
NW = 32
PPW = P // NW
CH = 8192
NCH = PPW // CH
NT = G * NCH
L = 16
VI = CH // L

_mesh = plsc.VectorSubcoreMesh(core_axis_name="c", subcore_axis_name="s")

_f32 = jnp.float32


@functools.partial(
    pl.kernel,
    out_type=(
        jax.ShapeDtypeStruct((G * P,), _f32),
        jax.ShapeDtypeStruct((G * P,), _f32),
        jax.ShapeDtypeStruct((NW, L), _f32),
    ),
    mesh=_mesh,
    scratch_types=[
        pltpu.VMEM((CH,), _f32),
        pltpu.VMEM((CH,), _f32),
        pltpu.VMEM((CH,), _f32),
        pltpu.VMEM((CH,), _f32),
        pltpu.VMEM((CH,), _f32),
        pltpu.VMEM((CH,), _f32),
        pltpu.VMEM((CH,), _f32),
        pltpu.VMEM((CH,), _f32),
        pltpu.VMEM((CH,), _f32),
        pltpu.VMEM((L,), _f32),
        pltpu.SemaphoreType.DMA,
        pltpu.SemaphoreType.DMA,
        pltpu.SemaphoreType.DMA,
        pltpu.SemaphoreType.DMA,
    ],
    compiler_params=pltpu.CompilerParams(use_tc_tiling_on_sc=True),
)
def _sc_comm(x_hbm, maps_hbm, masks_hbm, part_hbm,
             ch0_0, ch0_1, ch1_0, ch1_1, om_0, om_1, mk_0, mk_1,
             ones_v, acc_v, isem0, isem1, osem0, osem1):
    wid = lax.axis_index("s") * 2 + lax.axis_index("c")
    base = wid * PPW
    ones = jnp.ones((L,), _f32)
    zeros = jnp.zeros((L,), _f32)
    half = jnp.full((L,), 0.5, _f32)

    ch0 = (ch0_0, ch0_1)
    ch1 = (ch1_0, ch1_1)
    om = (om_0, om_1)
    mk = (mk_0, mk_1)
    isem = (isem0, isem1)
    osem = (osem0, osem1)

    @plsc.parallel_loop(0, VI, unroll=4)
    def _fill(i):
        ones_v[pl.ds(i * L, L)] = ones

    chunks = [(g, ci) for g in range(G) for ci in range(NCH)]

    def issue_loads(t, slot):
        g, ci = chunks[t]
        i0 = (2 * g) * P + base + ci * CH
        i1 = (2 * g + 1) * P + base + ci * CH
        d0 = pltpu.async_copy(x_hbm.at[pl.ds(i0, CH)], ch0[slot], isem[slot])
        d1 = pltpu.async_copy(x_hbm.at[pl.ds(i1, CH)], ch1[slot], isem[slot])
        return (d0, d1)

    ld = {0: issue_loads(0, 0)}
    st = {}
    acc = zeros
    for t in range(NT):
        g, ci = chunks[t]
        slot = t % 2
        if t + 1 < NT:
            ld[1 - slot] = issue_loads(t + 1, 1 - slot)
        for d in ld[slot]:
            d.wait()
        if t >= 2:
            for d in st[slot]:
                d.wait()

        agent = g % N
        is_even = (agent % 2) == 0
        is_rate = agent == 0
        c0r, c1r, omr, mkr = ch0[slot], ch1[slot], om[slot], mk[slot]

        if is_rate:
            @plsc.parallel_loop(0, VI, unroll=4, carry=acc)
            def _rate_body(i, a, c0r=c0r, c1r=c1r, omr=omr):
                sl = pl.ds(i * L, L)
                m = jnp.maximum(c0r[sl], c1r[sl])
                s = ones / (ones + jnp.exp(-m))
                omr[sl] = s
                return a + jnp.where(s > half, ones, zeros)
            acc = _rate_body
        elif is_even:
            @plsc.parallel_loop(0, VI, unroll=4)
            def _even_body(i, c0r=c0r, c1r=c1r, omr=omr):
                sl = pl.ds(i * L, L)
                m = jnp.maximum(c0r[sl], c1r[sl])
                omr[sl] = ones / (ones + jnp.exp(-m))
        else:
            @plsc.parallel_loop(0, VI, unroll=4)
            def _odd_body(i, c0r=c0r, c1r=c1r, omr=omr, mkr=mkr):
                sl = pl.ds(i * L, L)
                m = jnp.maximum(c0r[sl], c1r[sl])
                s = ones / (ones + jnp.exp(-m))
                mb = jnp.where(s > half, ones, zeros)
                omr[sl] = s * mb
                mkr[sl] = mb

        ob = g * P + base + ci * CH
        d_om = pltpu.async_copy(omr, maps_hbm.at[pl.ds(ob, CH)], osem[slot])
        mk_src = ones_v if is_even else mkr
        d_mk = pltpu.async_copy(mk_src, masks_hbm.at[pl.ds(ob, CH)], osem[slot])
        st[slot] = (d_om, d_mk)

    for slot in st:
        for d in st[slot]:
            d.wait()

    acc_v[...] = acc
    pltpu.sync_copy(acc_v, part_hbm.at[wid])


def kernel(batch_confidence_maps, record_len, pairwise_t_matrix):
    x = batch_confidence_maps.reshape(G * C * P)
    maps_flat, masks_flat, partials = _sc_comm(x)
    comm_maps = maps_flat.reshape(B, N, 1, H, W)
    comm_masks = masks_flat.reshape(G, 1, H, W)
    rate = jnp.sum(partials) / (B * H * W)
    return (comm_maps, comm_masks, rate)


# speedup vs baseline: 3.3616x; 2.4944x over previous
import functools

import jax
import jax.numpy as jnp
from jax import lax
from jax.experimental import pallas as pl
from jax.experimental.pallas import tpu as pltpu
from jax.experimental.pallas import tpu_sc as plsc

B, N, C, H, W = 2, 5, 2, 1024, 1024
G = B * N
NW = 32
RPW = H // NW
RC = 8
NCH = RPW // RC
NT = G * NCH
L = 16
VR = W // L

_mesh = plsc.VectorSubcoreMesh(core_axis_name="c", subcore_axis_name="s")

_f32 = jnp.float32


@functools.partial(
    pl.kernel,
    out_type=(
        jax.ShapeDtypeStruct((B, N, 1, H, W), _f32),
        jax.ShapeDtypeStruct((G, 1, H, W), _f32),
        jax.ShapeDtypeStruct((NW * L,), _f32),
    ),
    mesh=_mesh,
    scratch_types=[
        pltpu.VMEM((RC, W), _f32),
        pltpu.VMEM((RC, W), _f32),
        pltpu.VMEM((RC, W), _f32),
        pltpu.VMEM((RC, W), _f32),
        pltpu.VMEM((RC, W), _f32),
        pltpu.VMEM((RC, W), _f32),
        pltpu.VMEM((RC, W), _f32),
        pltpu.VMEM((RC, W), _f32),
        pltpu.VMEM((RC, W), _f32),
        pltpu.VMEM((L,), _f32),
        pltpu.SemaphoreType.DMA,
        pltpu.SemaphoreType.DMA,
        pltpu.SemaphoreType.DMA,
        pltpu.SemaphoreType.DMA,
    ],
    compiler_params=pltpu.CompilerParams(use_tc_tiling_on_sc=True),
)
def _sc_comm(x_hbm, maps_hbm, masks_hbm, part_hbm,
             ch0_0, ch0_1, ch1_0, ch1_1, om_0, om_1, mk_0, mk_1,
             ones_v, acc_v, isem0, isem1, osem0, osem1):
    wid = lax.axis_index("s") * 2 + lax.axis_index("c")
    row0 = wid * RPW
    ones = jnp.ones((L,), _f32)
    zeros = jnp.zeros((L,), _f32)
    half = jnp.full((L,), 0.5, _f32)

    ch0 = (ch0_0, ch0_1)
    ch1 = (ch1_0, ch1_1)
    om = (om_0, om_1)
    mk = (mk_0, mk_1)
    isem = (isem0, isem1)
    osem = (osem0, osem1)

    @plsc.parallel_loop(0, RC * VR, unroll=4)
    def _fill(i):
        ones_v[i >> 6, pl.ds((i & 63) * L, L)] = ones

    chunks = [(g, ci) for g in range(G) for ci in range(NCH)]

    def issue_loads(t, slot):
        g, ci = chunks[t]
        b, n = divmod(g, N)
        r = row0 + ci * RC
        d0 = pltpu.async_copy(
            x_hbm.at[b, n, 0, pl.ds(r, RC), :], ch0[slot], isem[slot])
        d1 = pltpu.async_copy(
            x_hbm.at[b, n, 1, pl.ds(r, RC), :], ch1[slot], isem[slot])
        return (d0, d1)

    ld = {0: issue_loads(0, 0)}
    st = {}
    acc = zeros
    for t in range(NT):
        g, ci = chunks[t]
        b, n = divmod(g, N)
        slot = t % 2
        if t + 1 < NT:
            ld[1 - slot] = issue_loads(t + 1, 1 - slot)
        for d in ld[slot]:
            d.wait()
        if t >= 2:
            for d in st[slot]:
                d.wait()

        is_even = (n % 2) == 0
        is_rate = n == 0
        c0r, c1r, omr, mkr = ch0[slot], ch1[slot], om[slot], mk[slot]

        if is_rate:
            @plsc.parallel_loop(0, RC * VR, unroll=4, carry=acc)
            def _rate_body(i, a, c0r=c0r, c1r=c1r, omr=omr):
                r = i >> 6
                sl = pl.ds((i & 63) * L, L)
                m = jnp.maximum(c0r[r, sl], c1r[r, sl])
                s = ones / (ones + jnp.exp(-m))
                omr[r, sl] = s
                return a + jnp.where(s > half, ones, zeros)
            acc = _rate_body
        elif is_even:
            @plsc.parallel_loop(0, RC * VR, unroll=4)
            def _even_body(i, c0r=c0r, c1r=c1r, omr=omr):
                r = i >> 6
                sl = pl.ds((i & 63) * L, L)
                m = jnp.maximum(c0r[r, sl], c1r[r, sl])
                omr[r, sl] = ones / (ones + jnp.exp(-m))
        else:
            @plsc.parallel_loop(0, RC * VR, unroll=4)
            def _odd_body(i, c0r=c0r, c1r=c1r, omr=omr, mkr=mkr):
                r = i >> 6
                sl = pl.ds((i & 63) * L, L)
                m = jnp.maximum(c0r[r, sl], c1r[r, sl])
                s = ones / (ones + jnp.exp(-m))
                mb = jnp.where(s > half, ones, zeros)
                omr[r, sl] = s * mb
                mkr[r, sl] = mb

        r = row0 + ci * RC
        d_om = pltpu.async_copy(
            omr, maps_hbm.at[b, n, 0, pl.ds(r, RC), :], osem[slot])
        mk_src = ones_v if is_even else mkr
        d_mk = pltpu.async_copy(
            mk_src, masks_hbm.at[g, 0, pl.ds(r, RC), :], osem[slot])
        st[slot] = (d_om, d_mk)

    for slot in st:
        for d in st[slot]:
            d.wait()

    acc_v[...] = acc
    pltpu.sync_copy(acc_v, part_hbm.at[pl.ds(wid * L, L)])


def kernel(batch_confidence_maps, record_len, pairwise_t_matrix):
    comm_maps, comm_masks, partials = _sc_comm(batch_confidence_maps)
    rate = jnp.sum(partials) / (B * H * W)
    return (comm_maps, comm_masks, rate)
